# baseline (speedup 1.0000x reference)
import functools

import numpy as np
import jax
import jax.numpy as jnp
from jax import lax
from jax.experimental import pallas as pl
from jax.experimental.pallas import tpu as pltpu
from jax.experimental.pallas import tpu_sc as plsc

_PS, _N, _B, _C, _H, _W = 8, 256, 8, 3, 512, 512
_NT = 32
_PPT = (_B * _N) // _NT
_CPP = 48
_NDMA = (_PPT * _CPP) // 128
_V = (_B * _C * _H * _W) // 16
_NP = _C * _PS * _PS


def _np_threefry2x32(k1, k2, c1, c2):
    x0 = c1.astype(np.uint32)
    x1 = c2.astype(np.uint32)
    ks0 = np.uint32(k1)
    ks1 = np.uint32(k2)
    ks2 = np.uint32(ks0 ^ ks1 ^ np.uint32(0x1BD11BDA))
    ks = (ks0, ks1, ks2)
    rots = ((13, 15, 26, 6), (17, 29, 16, 24))
    x0 = x0 + ks0
    x1 = x1 + ks1
    for i in range(5):
        for r in rots[i % 2]:
            x0 = x0 + x1
            x1 = (x1 << np.uint32(r)) | (x1 >> np.uint32(32 - r))
            x1 = x1 ^ x0
        x0 = x0 + ks[(i + 1) % 3]
        x1 = x1 + ks[(i + 2) % 3] + np.uint32(i + 1)
    return x0, x1


def _np_split(key, num):
    b1, b2 = _np_threefry2x32(
        key[0], key[1], np.zeros(num, np.uint32), np.arange(num, dtype=np.uint32)
    )
    return [(b1[i], b2[i]) for i in range(num)]


def _np_random_bits(key, shape):
    size = int(np.prod(shape))
    b1, b2 = _np_threefry2x32(
        key[0], key[1], np.zeros(size, np.uint32), np.arange(size, dtype=np.uint32)
    )
    return (b1 ^ b2).reshape(shape)


def _np_randint(key, shape, minval, maxval):
    k1, k2 = _np_split(key, 2)
    hi_bits = _np_random_bits(k1, shape)
    lo_bits = _np_random_bits(k2, shape)
    span = np.uint32(maxval - minval)
    mult = np.uint32((((2 ** 16) % int(span)) ** 2) % int(span))
    off = ((hi_bits % span) * mult + (lo_bits % span)) % span
    return (np.int32(minval) + off.astype(np.int32)).astype(np.int32)


def _make_coords():
    ck = (np.uint32(0), np.uint32(1))
    k1, k2, k3, k4 = _np_split(ck, 4)
    hi = _H - _PS + 1
    return tuple(_np_randint(k, (_N, _B), 0, hi) for k in (k1, k2, k3, k4))


def _gather_plan(ys, xs):
    p = np.arange(_B * _N)
    b, n = p // _N, p % _N
    y, x = ys[n, b].astype(np.int64), xs[n, b].astype(np.int64)
    k = np.arange(_CPP)
    r, which = k >> 1, k & 1
    c, dy = r >> 3, r & 7
    o = ((b[:, None] * 3 + c[None, :]) * _H + (y[:, None] + dy[None, :])) * _W + x[:, None]
    q = o >> 4
    phase = (x & 15).astype(np.int32)
    q = q + which[None, :] * (phase[:, None] > 8)
    idx = q.astype(np.int32).reshape(_NT, _NDMA, 128)
    ph = np.repeat(phase.reshape(_NT, _PPT), 16, axis=1)
    return idx, ph


_GY, _GX, _TY, _TX = _make_coords()
_IDX_G, _PH_G = _gather_plan(_GY, _GX)
_IDX_T, _PH_T = _gather_plan(_TY, _TX)

@functools.cache
def _build_sc():
    mesh = plsc.VectorSubcoreMesh(
        core_axis_name="c", subcore_axis_name="s", num_cores=2, num_subcores=16
    )
    return functools.partial(
        pl.kernel,
        out_type=jax.ShapeDtypeStruct((_NT, 32), jnp.float32),
        mesh=mesh,
        scratch_types=[
            pltpu.VMEM((_NDMA * 128, 16), jnp.float32),
            pltpu.VMEM((_NDMA, 128), jnp.int32),
            pltpu.VMEM((_PPT * 16,), jnp.int32),
            pltpu.VMEM((_PPT * 16,), jnp.float32),
            pltpu.VMEM((_PPT * 16,), jnp.float32),
            pltpu.VMEM((_PPT * 16,), jnp.float32),
            pltpu.VMEM((_PPT * 16,), jnp.float32),
            pltpu.VMEM((32,), jnp.float32),
            pltpu.SemaphoreType.DMA,
        ],
        compiler_params=pltpu.CompilerParams(
            needs_layout_passes=False, use_tc_tiling_on_sc=False
        ),
    )(_sc_loss)


def _sc_loss(gen_tab, tgt_tab, idx_g, idx_t, ph_g, ph_t, out_hbm,
             stag, idxv, phv_ref, accs_g, accq_g, accs_t, accq_t, outb, sem):
    wid = lax.axis_index("s") * 2 + lax.axis_index("c")
    iota = lax.broadcasted_iota(jnp.int32, (16,), 0)

    def process(tab, idx_hbm, ph_hbm, accs, accq):
        pltpu.sync_copy(idx_hbm.at[wid], idxv)
        pltpu.sync_copy(ph_hbm.at[wid], phv_ref)
        copies = [
            pltpu.async_copy(tab.at[idxv.at[j]], stag.at[pl.ds(j * 128, 128)], sem)
            for j in range(_NDMA)
        ]
        for cp in copies:
            cp.wait()

        def body(j, carry):
            phv = phv_ref[pl.ds(j * 16, 16)]
            m0 = (iota >= phv) & (iota < phv + 8)
            m1 = iota < phv - 8
            fbase = j * _CPP
            acc_s = jnp.zeros((16,), jnp.float32)
            acc_q = jnp.zeros((16,), jnp.float32)
            zero = jnp.zeros((16,), jnp.float32)
            for k in range(_CPP // 2):
                v0 = jnp.where(m0, stag[fbase + 2 * k], zero)
                v1 = jnp.where(m1, stag[fbase + 2 * k + 1], zero)
                acc_s = acc_s + v0 + v1
                acc_q = acc_q + v0 * v0 + v1 * v1
            accs[pl.ds(j * 16, 16)] = acc_s
            accq[pl.ds(j * 16, 16)] = acc_q
            return carry

        lax.fori_loop(0, _PPT, body, 0)

    process(gen_tab, idx_g, ph_g, accs_g, accq_g)
    process(tgt_tab, idx_t, ph_t, accs_t, accq_t)

    base16 = jnp.left_shift(iota, 4)
    lm = jnp.zeros((16,), jnp.float32)
    lv = jnp.zeros((16,), jnp.float32)
    for grp in range(_PPT // 16):
        def red(acc):
            tot = plsc.load_gather(acc, [base16 + grp * 256])
            for i in range(1, 16):
                tot = tot + plsc.load_gather(acc, [base16 + (grp * 256 + i)])
            return tot

        gs, gq = red(accs_g), red(accq_g)
        ts, tq = red(accs_t), red(accq_t)
        inv_n = 1.0 / float(_NP)
        inv_n1 = 1.0 / float(_NP - 1)
        gm = gs * inv_n
        tm = ts * inv_n
        gv = (gq - gs * gm) * inv_n1
        tv = (tq - ts * tm) * inv_n1
        dm = gm - tm
        dv = gv - tv
        lm = lm + dm * dm
        lv = lv + dv * dv
    outb[pl.ds(0, 16)] = lm
    outb[pl.ds(16, 16)] = lv
    pltpu.sync_copy(outb, out_hbm.at[wid])


def kernel(generated, target):
    gtab = generated.reshape(_V, 16)
    ttab = target.reshape(_V, 16)
    out = _build_sc()(gtab, ttab, _IDX_G, _IDX_T, _PH_G, _PH_T)
    denom = float(_B * _N)
    return out[:, :16].sum() / denom + out[:, 16:].sum() / denom
